# baseline (speedup 1.0000x reference)
import jax
import jax.numpy as jnp
from jax.experimental import pallas as pl
from jax.experimental.pallas import tpu as pltpu

_NUM_EXPERTS = 8
_HIDDEN = 1024
_INTER = 1024
_T = 64
_TOPK = 2
_P = 2
_EG = _NUM_EXPERTS // _P


def _moe_kernel(sel_ref, x_ref, g_ref, u_ref, dn_ref, out_ref):
    p = pl.program_id(0)
    j = pl.program_id(1)
    e = p * _EG + j

    @pl.when(j == 0)
    def _init():
        out_ref[...] = jnp.zeros_like(out_ref)

    x = x_ref[...]
    g = jax.lax.dot_general(
        x, g_ref[0], (((1,), (1,)), ((), ())),
        preferred_element_type=jnp.float32)
    u = jax.lax.dot_general(
        x, u_ref[0], (((1,), (1,)), ((), ())),
        preferred_element_type=jnp.float32)
    inter = g * jax.nn.sigmoid(g) * u
    o = jax.lax.dot_general(
        inter, dn_ref[0], (((1,), (1,)), ((), ())),
        preferred_element_type=jnp.float32)
    sel = sel_ref[...]
    for k in range(_TOPK):
        mk = sel[:, k:k + 1] == e
        cur = out_ref[0, :, k * _HIDDEN:(k + 1) * _HIDDEN]
        out_ref[0, :, k * _HIDDEN:(k + 1) * _HIDDEN] = jnp.where(mk, o, cur)


def kernel(hidden_states, selected_experts, gate_up_proj, down_proj):
    out_slabs = pl.pallas_call(
        _moe_kernel,
        grid=(_P, _EG),
        in_specs=[
            pl.BlockSpec((_T, _TOPK), lambda p, j: (0, 0)),
            pl.BlockSpec((_T, _HIDDEN), lambda p, j: (0, 0)),
            pl.BlockSpec((1, _INTER, _HIDDEN), lambda p, j: (p * _EG + j, 0, 0)),
            pl.BlockSpec((1, _INTER, _HIDDEN), lambda p, j: (p * _EG + j, 1, 0)),
            pl.BlockSpec((1, _HIDDEN, _INTER), lambda p, j: (p * _EG + j, 0, 0)),
        ],
        out_specs=pl.BlockSpec((1, _T, _TOPK * _HIDDEN), lambda p, j: (p, 0, 0)),
        out_shape=jax.ShapeDtypeStruct((_P, _T, _TOPK * _HIDDEN), jnp.float32),
        compiler_params=pltpu.CompilerParams(
            dimension_semantics=("parallel", "arbitrary")),
    )(selected_experts, hidden_states, gate_up_proj, gate_up_proj, down_proj)
    return (out_slabs[0] + out_slabs[1]).reshape(_T, _TOPK, _HIDDEN)
